# baseline (speedup 1.0000x reference)
import jax
import jax.numpy as jnp
from jax import lax
from jax.experimental import pallas as pl
from jax.experimental.pallas import tpu as pltpu
from jax.experimental.pallas import tpu_sc as plsc

NC = 2
NS = 16
L = 16
NW = NC * NS

B = 128
N = 32768
D = 8
K = 128

ROWS_PER_W = B // NW
SEG = 256
NSEG = N // SEG
VPS = SEG // L
LAG = 16

_BIG = 2**30


def _sc_body(tokens_hbm, maskf_hbm, out_tok_hbm, out_msk_hbm,
             keys_v, bt0, bt1, bm0, bm1, seg_max, vals_v,
             msk_out_v, winners_v, idx_s, st0, st1, sm0, sm1, gsem):
    wid = lax.axis_index("s") * NC + lax.axis_index("c")
    lane = lax.iota(jnp.int32, L)
    lane0 = lane == 0
    col2 = jnp.full((L,), 2, jnp.int32)

    def bf(x):
        return jnp.full((L,), x, jnp.float32)

    def bi(x):
        return jnp.full((L,), x, jnp.int32)

    def process_row(r, carry0):
        row = wid * ROWS_PER_W + r

        def issue(c, bt, bm, st, sm):
            pltpu.async_copy(
                tokens_hbm.at[row, pl.ds(c * SEG, SEG), :], bt, st)
            pltpu.async_copy(
                maskf_hbm.at[row, pl.ds(c * SEG, SEG)], bm, sm)

        def wait(bt, bm, st, sm):
            pltpu.make_async_copy(
                tokens_hbm.at[row, pl.ds(0, SEG), :], bt, st).wait()
            pltpu.make_async_copy(
                maskf_hbm.at[row, pl.ds(0, SEG)], bm, sm).wait()

        def compute(c, bt, bm):
            m = bf(-3.0)
            for t in range(VPS):
                off = t * L
                ch2 = plsc.load_gather(bt, [off + lane, col2])
                mv = bm[pl.ds(off, L)]
                key = jnp.where(mv > 0.5, bf(-1.0), jnp.abs(ch2))
                keys_v[pl.ds(c * SEG + off, L)] = key
                m = jnp.maximum(m, key)
            plsc.store_scatter(
                seg_max, [bi(c)], jnp.full((L,), jnp.max(m)), mask=lane0)

        issue(0, bt0, bm0, st0, sm0)
        issue(1, bt1, bm1, st1, sm1)

        def pair_body(p, carry):
            c0 = p * 2
            wait(bt0, bm0, st0, sm0)
            compute(c0, bt0, bm0)

            @pl.when(p < NSEG // 2 - 1)
            def _():
                issue(c0 + 2, bt0, bm0, st0, sm0)

            wait(bt1, bm1, st1, sm1)
            compute(c0 + 1, bt1, bm1)

            @pl.when(p < NSEG // 2 - 1)
            def _():
                issue(c0 + 3, bt1, bm1, st1, sm1)

            return carry

        lax.fori_loop(0, NSEG // 2, pair_body, 0)

        def extract(j, carry):
            m = seg_max[pl.ds(0, L)]
            for i in range(1, NSEG // L):
                m = jnp.maximum(m, seg_max[pl.ds(i * L, L)])
            M = jnp.max(m)

            sstar = jnp.int32(_BIG)
            for i in range(NSEG // L):
                sv = seg_max[pl.ds(i * L, L)]
                cand = jnp.min(jnp.where(sv == M, i * L + lane, _BIG))
                sstar = jnp.minimum(sstar, cand)

            base = sstar * SEG
            eidx = jnp.int32(_BIG)
            for t in range(VPS):
                kv = keys_v[pl.ds(base + t * L, L)]
                cand = jnp.min(
                    jnp.where(kv == M, base + t * L + lane, _BIG))
                eidx = jnp.minimum(eidx, cand)

            idx_s[j] = eidx
            plsc.store_scatter(vals_v, [bi(j)], jnp.full((L,), M),
                               mask=lane0)
            plsc.store_scatter(keys_v, [jnp.full((L,), eidx)],
                               bf(-2.0), mask=lane0)

            m2 = keys_v[pl.ds(base, L)]
            for t in range(1, VPS):
                m2 = jnp.maximum(m2, keys_v[pl.ds(base + t * L, L)])
            plsc.store_scatter(seg_max, [jnp.full((L,), sstar)],
                               jnp.full((L,), jnp.max(m2)), mask=lane0)
            return carry

        lax.fori_loop(0, K, extract, 0)

        def fetch(j, carry):
            @pl.when(j < K)
            def _():
                e = idx_s[j]
                pltpu.async_copy(
                    tokens_hbm.at[row, pl.ds(e, 1), :],
                    winners_v.at[pl.ds(j, 1), :], gsem)

            @pl.when(j >= LAG)
            def _():
                pltpu.make_async_copy(
                    tokens_hbm.at[row, pl.ds(0, 1), :],
                    winners_v.at[pl.ds(0, 1), :], gsem).wait()

            return carry

        lax.fori_loop(0, K + LAG, fetch, 0)
        pltpu.sync_copy(winners_v, out_tok_hbm.at[row])

        def mk(i, carry):
            v = vals_v[pl.ds(i * L, L)]
            msk_out_v[pl.ds(i * L, L)] = jnp.where(
                v == -1.0, bf(1.0), bf(0.0))
            return carry

        lax.fori_loop(0, K // L, mk, 0)
        pltpu.sync_copy(msk_out_v, out_msk_hbm.at[row])
        return carry0

    lax.fori_loop(0, ROWS_PER_W, process_row, 0)


@jax.jit
def _run(tokens, maskf):
    mesh = plsc.VectorSubcoreMesh(
        core_axis_name="c", subcore_axis_name="s",
        num_cores=NC, num_subcores=NS)
    f = pl.kernel(
        _sc_body,
        out_type=(
            jax.ShapeDtypeStruct((B, K, D), jnp.float32),
            jax.ShapeDtypeStruct((B, K), jnp.float32),
        ),
        mesh=mesh,
        compiler_params=pltpu.CompilerParams(needs_layout_passes=False),
        scratch_types=[
            pltpu.VMEM((N,), jnp.float32),
            pltpu.VMEM((SEG, D), jnp.float32),
            pltpu.VMEM((SEG, D), jnp.float32),
            pltpu.VMEM((SEG,), jnp.float32),
            pltpu.VMEM((SEG,), jnp.float32),
            pltpu.VMEM((NSEG,), jnp.float32),
            pltpu.VMEM((K,), jnp.float32),
            pltpu.VMEM((K,), jnp.float32),
            pltpu.VMEM((K, D), jnp.float32),
            pltpu.SMEM((K,), jnp.int32),
            pltpu.SemaphoreType.DMA,
            pltpu.SemaphoreType.DMA,
            pltpu.SemaphoreType.DMA,
            pltpu.SemaphoreType.DMA,
            pltpu.SemaphoreType.DMA,
        ],
    )
    return f(tokens, maskf)


def kernel(tokens, obs_mask):
    maskf = obs_mask.astype(jnp.float32)
    out_tok, mask_f = _run(tokens, maskf)
    return out_tok, mask_f != 0.0


# speedup vs baseline: 5.1544x; 5.1544x over previous
import jax
import jax.numpy as jnp
from jax import lax
from jax.experimental import pallas as pl
from jax.experimental.pallas import tpu as pltpu
from jax.experimental.pallas import tpu_sc as plsc

NC = 2
NS = 16
L = 16
NW = NC * NS

B = 128
N = 32768
D = 8
K = 128

TPG = 128 // D
NG = N // TPG
ROWS_PER_W = B // NW
TPC = 2048
GPC = TPC // TPG
NCH = N // TPC
SEG = 256
NSEG = N // SEG
SPC = TPC // SEG
LAG = 16

_BIG = 2**30


def _sc_body(tokens_hbm, maskf_hbm, out_tok_hbm, out_msk_hbm,
             keys_v, bt0, bt1, bm0, bm1, seg_max, idx_v, vals_v,
             msk_out_v, wgrp_v, rows_v, idx_s,
             st0, st1, sm0, sm1, gsem):
    wid = lax.axis_index("s") * NC + lax.axis_index("c")
    lane = lax.iota(jnp.int32, L)
    lane0 = lane == 0
    ch2_lane = lane * D + 2

    def bf(x):
        return jnp.full((L,), x, jnp.float32)

    def bi(x):
        return jnp.full((L,), x, jnp.int32)

    def process_row(r, carry0):
        row = wid * ROWS_PER_W + r

        def issue(c, bt, bm, st, sm):
            pltpu.async_copy(
                tokens_hbm.at[row, pl.ds(c * GPC, GPC), :], bt, st)
            pltpu.async_copy(
                maskf_hbm.at[row, pl.ds(c * TPC, TPC)], bm, sm)

        def wait(bt, bm, st, sm):
            pltpu.make_async_copy(
                tokens_hbm.at[row, pl.ds(0, GPC), :], bt, st).wait()
            pltpu.make_async_copy(
                maskf_hbm.at[row, pl.ds(0, TPC)], bm, sm).wait()

        def compute(c, bt, bm):
            def seg_body(s, carry1):
                base = s * SEG
                m = bf(-3.0)
                for t in range(SEG // L):
                    off = base + t * L
                    ch2 = plsc.load_gather(bt, [bi(off // TPG), ch2_lane])
                    mv = bm[pl.ds(off, L)]
                    key = jnp.where(mv > 0.5, bf(-1.0), jnp.abs(ch2))
                    keys_v[pl.ds(c * TPC + off, L)] = key
                    m = jnp.maximum(m, key)
                plsc.store_scatter(
                    seg_max, [bi(c * SPC + s)],
                    jnp.full((L,), jnp.max(m)), mask=lane0)
                return carry1

            lax.fori_loop(0, SPC, seg_body, 0)

        issue(0, bt0, bm0, st0, sm0)
        issue(1, bt1, bm1, st1, sm1)

        def pair_body(p, carry):
            c0 = p * 2
            wait(bt0, bm0, st0, sm0)
            compute(c0, bt0, bm0)

            @pl.when(p < NCH // 2 - 1)
            def _():
                issue(c0 + 2, bt0, bm0, st0, sm0)

            wait(bt1, bm1, st1, sm1)
            compute(c0 + 1, bt1, bm1)

            @pl.when(p < NCH // 2 - 1)
            def _():
                issue(c0 + 3, bt1, bm1, st1, sm1)

            return carry

        lax.fori_loop(0, NCH // 2, pair_body, 0)

        def extract(j, carry):
            m = seg_max[pl.ds(0, L)]
            for i in range(1, NSEG // L):
                m = jnp.maximum(m, seg_max[pl.ds(i * L, L)])
            M = jnp.max(m)

            sstar = jnp.int32(_BIG)
            for i in range(NSEG // L):
                sv = seg_max[pl.ds(i * L, L)]
                cand = jnp.min(jnp.where(sv == M, i * L + lane, _BIG))
                sstar = jnp.minimum(sstar, cand)

            base = sstar * SEG
            eidx = jnp.int32(_BIG)
            for t in range(SEG // L):
                kv = keys_v[pl.ds(base + t * L, L)]
                cand = jnp.min(
                    jnp.where(kv == M, base + t * L + lane, _BIG))
                eidx = jnp.minimum(eidx, cand)

            idx_s[j] = eidx
            plsc.store_scatter(idx_v, [bi(j)], jnp.full((L,), eidx),
                               mask=lane0)
            plsc.store_scatter(vals_v, [bi(j)], jnp.full((L,), M),
                               mask=lane0)
            plsc.store_scatter(keys_v, [jnp.full((L,), eidx)],
                               bf(-2.0), mask=lane0)

            m2 = keys_v[pl.ds(base, L)]
            for t in range(1, SEG // L):
                m2 = jnp.maximum(m2, keys_v[pl.ds(base + t * L, L)])
            plsc.store_scatter(seg_max, [jnp.full((L,), sstar)],
                               jnp.full((L,), jnp.max(m2)), mask=lane0)
            return carry

        lax.fori_loop(0, K, extract, 0)

        def fetch(j, carry):
            @pl.when(j < K)
            def _():
                e = idx_s[j]
                pltpu.async_copy(
                    tokens_hbm.at[row, pl.ds(e // TPG, 1), :],
                    wgrp_v.at[pl.ds(j, 1), :], gsem)

            @pl.when(j >= LAG)
            def _():
                pltpu.make_async_copy(
                    tokens_hbm.at[row, pl.ds(0, 1), :],
                    wgrp_v.at[pl.ds(0, 1), :], gsem).wait()

            return carry

        lax.fori_loop(0, K + LAG, fetch, 0)

        jo = lane >> 3
        do = lane & 7

        def ex(i, carry):
            jv = i * 2 + jo
            tsel = plsc.load_gather(idx_v, [jv])
            col = (tsel & (TPG - 1)) * D + do
            rows_v[pl.ds(i * L, L)] = plsc.load_gather(wgrp_v, [jv, col])
            return carry

        lax.fori_loop(0, K * D // L, ex, 0)
        pltpu.sync_copy(rows_v, out_tok_hbm.at[row])

        def mk(i, carry):
            v = vals_v[pl.ds(i * L, L)]
            msk_out_v[pl.ds(i * L, L)] = jnp.where(
                v == -1.0, bf(1.0), bf(0.0))
            return carry

        lax.fori_loop(0, K // L, mk, 0)
        pltpu.sync_copy(msk_out_v, out_msk_hbm.at[row])
        return carry0

    lax.fori_loop(0, ROWS_PER_W, process_row, 0)


@jax.jit
def _run(tokens3, maskf):
    mesh = plsc.VectorSubcoreMesh(
        core_axis_name="c", subcore_axis_name="s",
        num_cores=NC, num_subcores=NS)
    f = pl.kernel(
        _sc_body,
        out_type=(
            jax.ShapeDtypeStruct((B, K * D), jnp.float32),
            jax.ShapeDtypeStruct((B, K), jnp.float32),
        ),
        mesh=mesh,
        compiler_params=pltpu.CompilerParams(needs_layout_passes=False),
        scratch_types=[
            pltpu.VMEM((N,), jnp.float32),
            pltpu.VMEM((GPC, 128), jnp.float32),
            pltpu.VMEM((GPC, 128), jnp.float32),
            pltpu.VMEM((TPC,), jnp.float32),
            pltpu.VMEM((TPC,), jnp.float32),
            pltpu.VMEM((NSEG,), jnp.float32),
            pltpu.VMEM((K,), jnp.int32),
            pltpu.VMEM((K,), jnp.float32),
            pltpu.VMEM((K,), jnp.float32),
            pltpu.VMEM((K, 128), jnp.float32),
            pltpu.VMEM((K * D,), jnp.float32),
            pltpu.SMEM((K,), jnp.int32),
            pltpu.SemaphoreType.DMA,
            pltpu.SemaphoreType.DMA,
            pltpu.SemaphoreType.DMA,
            pltpu.SemaphoreType.DMA,
            pltpu.SemaphoreType.DMA,
        ],
    )
    return f(tokens3, maskf)


def kernel(tokens, obs_mask):
    tokens3 = tokens.reshape(B, NG, 128)
    maskf = obs_mask.astype(jnp.float32)
    out_tok, mask_f = _run(tokens3, maskf)
    return out_tok.reshape(B, K, D), mask_f != 0.0
